# Initial kernel scaffold; baseline (speedup 1.0000x reference)
import functools

import jax
import jax.numpy as jnp
from jax import lax
from jax.experimental import pallas as pl
from jax.experimental.pallas import tpu as pltpu
from jax.experimental.pallas import tpu_sc as plsc

N = 10000
D = 128
E_MSG = 256000
E_TGT = 64000
C = 128
NTILES = 16
ROWS_PER_TILE = N // NTILES

_mesh = plsc.VectorSubcoreMesh(core_axis_name="c", subcore_axis_name="s")


def _sc_body(mei, tei, tw, x, zeros, agg_out,
             src_v, dst_v, rows_v, w_v, agg_sh, sem_g):
    c = lax.axis_index("c")
    s = lax.axis_index("s")
    w = s * 2 + c

    pltpu.sync_copy(zeros, agg_sh.at[pl.ds(s * ROWS_PER_TILE, ROWS_PER_TILE)])
    plsc.subcore_barrier()

    def do_chunk(b_edge, weighted):
        if weighted:
            pltpu.sync_copy(tei.at[1, pl.ds(b_edge, C)], src_v.at[0])
            pltpu.sync_copy(tei.at[0, pl.ds(b_edge, C)], dst_v.at[0])
            pltpu.sync_copy(tw.at[pl.ds(b_edge, C)], w_v)
        else:
            pltpu.sync_copy(mei.at[0, pl.ds(b_edge, C)], src_v.at[0])
            pltpu.sync_copy(mei.at[1, pl.ds(b_edge, C)], dst_v.at[0])
        pltpu.async_copy(x.at[src_v.at[0]], rows_v.at[0], sem_g).wait()
        if weighted:
            def scale_body(e, carry):
                ws = plsc.load_gather(w_v, [jnp.full((16,), 0, jnp.int32) + e])
                for k in range(D // 16):
                    rows_v[0, e, pl.ds(k * 16, 16)] = (
                        rows_v[0, e, pl.ds(k * 16, 16)] * ws)
                return carry
            lax.fori_loop(0, C, scale_body, 0)
        pltpu.sync_copy(rows_v.at[0], agg_sh.at[dst_v.at[0]], add=True)

    n_msg = jnp.where(w < 16, 63, 62)
    base_msg = w * 62 + jnp.minimum(w, 16)

    def msg_body(j, carry):
        do_chunk((base_msg + j) * C, weighted=False)
        return carry
    lax.fori_loop(0, n_msg, msg_body, 0)

    n_tgt = jnp.where(w < 20, 16, 15)
    base_tgt = w * 15 + jnp.minimum(w, 20)

    def tgt_body(j, carry):
        do_chunk((base_tgt + j) * C, weighted=True)
        return carry
    lax.fori_loop(0, n_tgt, tgt_body, 0)

    plsc.subcore_barrier()
    pltpu.sync_copy(agg_sh.at[pl.ds(s * ROWS_PER_TILE, ROWS_PER_TILE)],
                    agg_out.at[c, pl.ds(s * ROWS_PER_TILE, ROWS_PER_TILE)])


_sc_scatter = functools.partial(
    pl.kernel,
    out_type=jax.ShapeDtypeStruct((2, N, D), jnp.float32),
    mesh=_mesh,
    scratch_types=[
        pltpu.VMEM((2, C), jnp.int32),
        pltpu.VMEM((2, C), jnp.int32),
        pltpu.VMEM((2, C, D), jnp.float32),
        pltpu.VMEM((C,), jnp.float32),
        pltpu.VMEM_SHARED((N, D), jnp.float32),
        pltpu.SemaphoreType.DMA,
    ],
)(_sc_body)


ROWS_PER_BLK = 1000


def _dense_body(agg_ref, x_ref, wrel_ref, wroot_ref, wmu_ref,
                brel_ref, bmu_ref, out_ref):
    agg = agg_ref[0] + agg_ref[1]
    conv = jnp.dot(agg, wrel_ref[...], preferred_element_type=jnp.float32)
    conv = conv + jnp.dot(x_ref[...], wroot_ref[...],
                          preferred_element_type=jnp.float32)
    conv = conv + brel_ref[...]
    h = jnp.maximum(conv, 0.0)
    mu = jnp.dot(h, wmu_ref[...], preferred_element_type=jnp.float32)
    mu = mu + bmu_ref[...]
    out_ref[...] = jnp.maximum(mu, 0.0)


_dense = pl.pallas_call(
    _dense_body,
    grid=(N // ROWS_PER_BLK,),
    in_specs=[
        pl.BlockSpec((2, ROWS_PER_BLK, D), lambda i: (0, i, 0)),
        pl.BlockSpec((ROWS_PER_BLK, D), lambda i: (i, 0)),
        pl.BlockSpec((D, D), lambda i: (0, 0)),
        pl.BlockSpec((D, D), lambda i: (0, 0)),
        pl.BlockSpec((D, D), lambda i: (0, 0)),
        pl.BlockSpec((1, D), lambda i: (0, 0)),
        pl.BlockSpec((1, D), lambda i: (0, 0)),
    ],
    out_specs=pl.BlockSpec((ROWS_PER_BLK, D), lambda i: (i, 0)),
    out_shape=jax.ShapeDtypeStruct((N, D), jnp.float32),
)


def kernel(x, message_edge_index, target_edge_index, target_edge_weights,
           W_rel, b_rel, W_root, W_mu, b_mu, W_std, b_std):
    zeros = jnp.zeros((ROWS_PER_TILE, D), jnp.float32)
    agg2 = _sc_scatter(message_edge_index, target_edge_index,
                       target_edge_weights, x, zeros)
    out = _dense(agg2, x, W_rel, W_root, W_mu,
                 b_rel.reshape(1, D), b_mu.reshape(1, D))
    return (out, target_edge_weights)


# speedup vs baseline: 5.8685x; 5.8685x over previous
import functools

import jax
import jax.numpy as jnp
from jax import lax
from jax.experimental import pallas as pl
from jax.experimental.pallas import tpu as pltpu
from jax.experimental.pallas import tpu_sc as plsc

N = 10000
D = 128
E_MSG = 256000
E_TGT = 64000
C = 128
NTILES = 16
NPAD = 10240
ROWS_PER_TILE = NPAD // NTILES

_mesh = plsc.VectorSubcoreMesh(core_axis_name="c", subcore_axis_name="s")


def _sc_body(msg_src, msg_dst, tgt_src, tgt_dst, tw, x, zeros, agg_out,
             src_v, dst_v, rows_v, w_v, agg_sh, sem_g):
    c = lax.axis_index("c")
    s = lax.axis_index("s")
    w = s * 2 + c

    pltpu.sync_copy(zeros, agg_sh.at[pl.ds(s * ROWS_PER_TILE, ROWS_PER_TILE)])
    plsc.subcore_barrier()

    def do_chunk(b_edge, weighted):
        if weighted:
            pltpu.sync_copy(tgt_src.at[pl.ds(b_edge, C)], src_v.at[0])
            pltpu.sync_copy(tgt_dst.at[pl.ds(b_edge, C)], dst_v.at[0])
            pltpu.sync_copy(tw.at[pl.ds(b_edge, C)], w_v)
        else:
            pltpu.sync_copy(msg_src.at[pl.ds(b_edge, C)], src_v.at[0])
            pltpu.sync_copy(msg_dst.at[pl.ds(b_edge, C)], dst_v.at[0])
        pltpu.async_copy(x.at[src_v.at[0]], rows_v.at[0], sem_g).wait()
        if weighted:
            def scale_body(e, carry):
                wv = w_v[pl.ds((e // 16) * 16, 16)]
                oh = lax.iota(jnp.int32, 16) == (e % 16)
                ws = jnp.broadcast_to(jnp.sum(jnp.where(oh, wv, 0.0)), (16,))
                for k in range(D // 16):
                    rows_v[0, e, pl.ds(k * 16, 16)] = (
                        rows_v[0, e, pl.ds(k * 16, 16)] * ws)
                return carry
            lax.fori_loop(0, C, scale_body, 0)
        pltpu.sync_copy(rows_v.at[0], agg_sh.at[dst_v.at[0]], add=True)

    n_msg = jnp.where(w < 16, 63, 62)
    base_msg = w * 62 + jnp.minimum(w, 16)

    def msg_body(j, carry):
        do_chunk((base_msg + j) * C, weighted=False)
        return carry
    lax.fori_loop(0, n_msg, msg_body, 0)

    n_tgt = jnp.where(w < 20, 16, 15)
    base_tgt = w * 15 + jnp.minimum(w, 20)

    def tgt_body(j, carry):
        do_chunk((base_tgt + j) * C, weighted=True)
        return carry
    lax.fori_loop(0, n_tgt, tgt_body, 0)

    plsc.subcore_barrier()
    pltpu.sync_copy(agg_sh.at[pl.ds(s * ROWS_PER_TILE, ROWS_PER_TILE)],
                    agg_out.at[c, pl.ds(s * ROWS_PER_TILE, ROWS_PER_TILE)])


_sc_scatter = functools.partial(
    pl.kernel,
    out_type=jax.ShapeDtypeStruct((2, NPAD, D), jnp.float32),
    mesh=_mesh,
    compiler_params=pltpu.CompilerParams(needs_layout_passes=False),
    scratch_types=[
        pltpu.VMEM((2, C), jnp.int32),
        pltpu.VMEM((2, C), jnp.int32),
        pltpu.VMEM((2, C, D), jnp.float32),
        pltpu.VMEM((C,), jnp.float32),
        pltpu.VMEM_SHARED((NPAD, D), jnp.float32),
        pltpu.SemaphoreType.DMA,
    ],
)(_sc_body)


ROWS_PER_BLK = 1000


def _dense_body(agg_ref, x_ref, wrel_ref, wroot_ref, wmu_ref,
                brel_ref, bmu_ref, out_ref):
    agg = agg_ref[0] + agg_ref[1]
    conv = jnp.dot(agg, wrel_ref[...], preferred_element_type=jnp.float32)
    conv = conv + jnp.dot(x_ref[...], wroot_ref[...],
                          preferred_element_type=jnp.float32)
    conv = conv + brel_ref[...]
    h = jnp.maximum(conv, 0.0)
    mu = jnp.dot(h, wmu_ref[...], preferred_element_type=jnp.float32)
    mu = mu + bmu_ref[...]
    out_ref[...] = jnp.maximum(mu, 0.0)


_dense = pl.pallas_call(
    _dense_body,
    grid=(N // ROWS_PER_BLK,),
    in_specs=[
        pl.BlockSpec((2, ROWS_PER_BLK, D), lambda i: (0, i, 0)),
        pl.BlockSpec((ROWS_PER_BLK, D), lambda i: (i, 0)),
        pl.BlockSpec((D, D), lambda i: (0, 0)),
        pl.BlockSpec((D, D), lambda i: (0, 0)),
        pl.BlockSpec((D, D), lambda i: (0, 0)),
        pl.BlockSpec((1, D), lambda i: (0, 0)),
        pl.BlockSpec((1, D), lambda i: (0, 0)),
    ],
    out_specs=pl.BlockSpec((ROWS_PER_BLK, D), lambda i: (i, 0)),
    out_shape=jax.ShapeDtypeStruct((N, D), jnp.float32),
)


def kernel(x, message_edge_index, target_edge_index, target_edge_weights,
           W_rel, b_rel, W_root, W_mu, b_mu, W_std, b_std):
    zeros = jnp.zeros((ROWS_PER_TILE, D), jnp.float32)
    agg2 = _sc_scatter(message_edge_index[0], message_edge_index[1],
                       target_edge_index[1], target_edge_index[0],
                       target_edge_weights, x, zeros)
    out = _dense(agg2, x, W_rel, W_root, W_mu,
                 b_rel.reshape(1, D), b_mu.reshape(1, D))
    return (out, target_edge_weights)


# speedup vs baseline: 9.9030x; 1.6875x over previous
import functools

import jax
import jax.numpy as jnp
from jax import lax
from jax.experimental import pallas as pl
from jax.experimental.pallas import tpu as pltpu
from jax.experimental.pallas import tpu_sc as plsc

N = 10000
D = 128
E_MSG = 256000
E_TGT = 64000
C = 128
NTILES = 16
NPAD = 10240
ROWS_PER_TILE = NPAD // NTILES

_mesh = plsc.VectorSubcoreMesh(core_axis_name="c", subcore_axis_name="s")


def _sc_body(msg_src, msg_dst, tgt_src, tgt_dst, tw, x, zeros, agg_out,
             src_v, dst_v, rows_v, w_v, agg_sh, sem_i, sem_g, sem_s):
    c = lax.axis_index("c")
    s = lax.axis_index("s")
    w = s * 2 + c

    pltpu.sync_copy(zeros, agg_sh.at[pl.ds(s * ROWS_PER_TILE, ROWS_PER_TILE)])
    plsc.subcore_barrier()

    def run_seg(srcs, dsts, n, base, weighted):
        def idx_descs(j):
            b = (base + j) * C
            slot = j % 4
            ds = [pltpu.make_async_copy(srcs.at[pl.ds(b, C)],
                                        src_v.at[slot], sem_i),
                  pltpu.make_async_copy(dsts.at[pl.ds(b, C)],
                                        dst_v.at[slot], sem_i)]
            if weighted:
                ds.append(pltpu.make_async_copy(tw.at[pl.ds(b, C)],
                                                w_v.at[slot], sem_i))
            return ds

        def gather_desc(j):
            return pltpu.make_async_copy(x.at[src_v.at[j % 4]],
                                         rows_v.at[j % 2], sem_g)

        def scatter_desc(j):
            return pltpu.make_async_copy(rows_v.at[j % 2],
                                         agg_sh.at[dst_v.at[j % 4]], sem_s)

        def scale(j):
            rbuf = j % 2
            wbuf = j % 4

            def scale_body(e, carry):
                wv = w_v[wbuf, pl.ds((e // 16) * 16, 16)]
                oh = lax.iota(jnp.int32, 16) == (e % 16)
                ws = jnp.broadcast_to(jnp.sum(jnp.where(oh, wv, 0.0)), (16,))
                for k in range(D // 16):
                    rows_v[rbuf, e, pl.ds(k * 16, 16)] = (
                        rows_v[rbuf, e, pl.ds(k * 16, 16)] * ws)
                return carry
            lax.fori_loop(0, C, scale_body, 0)

        for d in idx_descs(0):
            d.start()
        for d in idx_descs(0):
            d.wait()
        gather_desc(0).start()

        @pl.when(n > 1)
        def _():
            for d in idx_descs(1):
                d.start()

        def body(j, carry):
            gather_desc(j).wait()

            @pl.when(j >= 1)
            def _():
                scatter_desc(j - 1).wait()

            @pl.when(j + 1 < n)
            def _():
                for d in idx_descs(j + 1):
                    d.wait()
                gather_desc(j + 1).start()

            @pl.when(j + 2 < n)
            def _():
                for d in idx_descs(j + 2):
                    d.start()

            if weighted:
                scale(j)
            scatter_desc(j).start(add=True)
            return carry

        lax.fori_loop(0, n, body, 0)
        scatter_desc(n - 1).wait()

    run_seg(msg_src, msg_dst, jnp.where(w < 16, 63, 62),
            w * 62 + jnp.minimum(w, 16), weighted=False)
    run_seg(tgt_src, tgt_dst, jnp.where(w < 20, 16, 15),
            w * 15 + jnp.minimum(w, 20), weighted=True)

    plsc.subcore_barrier()
    pltpu.sync_copy(agg_sh.at[pl.ds(s * ROWS_PER_TILE, ROWS_PER_TILE)],
                    agg_out.at[c, pl.ds(s * ROWS_PER_TILE, ROWS_PER_TILE)])


_sc_scatter = functools.partial(
    pl.kernel,
    out_type=jax.ShapeDtypeStruct((2, NPAD, D), jnp.float32),
    mesh=_mesh,
    compiler_params=pltpu.CompilerParams(needs_layout_passes=False),
    scratch_types=[
        pltpu.VMEM((4, C), jnp.int32),
        pltpu.VMEM((4, C), jnp.int32),
        pltpu.VMEM((2, C, D), jnp.float32),
        pltpu.VMEM((4, C), jnp.float32),
        pltpu.VMEM_SHARED((NPAD, D), jnp.float32),
        pltpu.SemaphoreType.DMA,
        pltpu.SemaphoreType.DMA,
        pltpu.SemaphoreType.DMA,
    ],
)(_sc_body)


ROWS_PER_BLK = 1000


def _dense_body(agg_ref, x_ref, wrel_ref, wroot_ref, wmu_ref,
                brel_ref, bmu_ref, out_ref):
    agg = agg_ref[0] + agg_ref[1]
    conv = jnp.dot(agg, wrel_ref[...], preferred_element_type=jnp.float32)
    conv = conv + jnp.dot(x_ref[...], wroot_ref[...],
                          preferred_element_type=jnp.float32)
    conv = conv + brel_ref[...]
    h = jnp.maximum(conv, 0.0)
    mu = jnp.dot(h, wmu_ref[...], preferred_element_type=jnp.float32)
    mu = mu + bmu_ref[...]
    out_ref[...] = jnp.maximum(mu, 0.0)


_dense = pl.pallas_call(
    _dense_body,
    grid=(N // ROWS_PER_BLK,),
    in_specs=[
        pl.BlockSpec((2, ROWS_PER_BLK, D), lambda i: (0, i, 0)),
        pl.BlockSpec((ROWS_PER_BLK, D), lambda i: (i, 0)),
        pl.BlockSpec((D, D), lambda i: (0, 0)),
        pl.BlockSpec((D, D), lambda i: (0, 0)),
        pl.BlockSpec((D, D), lambda i: (0, 0)),
        pl.BlockSpec((1, D), lambda i: (0, 0)),
        pl.BlockSpec((1, D), lambda i: (0, 0)),
    ],
    out_specs=pl.BlockSpec((ROWS_PER_BLK, D), lambda i: (i, 0)),
    out_shape=jax.ShapeDtypeStruct((N, D), jnp.float32),
)


def kernel(x, message_edge_index, target_edge_index, target_edge_weights,
           W_rel, b_rel, W_root, W_mu, b_mu, W_std, b_std):
    zeros = jnp.zeros((ROWS_PER_TILE, D), jnp.float32)
    agg2 = _sc_scatter(message_edge_index[0], message_edge_index[1],
                       target_edge_index[1], target_edge_index[0],
                       target_edge_weights, x, zeros)
    out = _dense(agg2, x, W_rel, W_root, W_mu,
                 b_rel.reshape(1, D), b_mu.reshape(1, D))
    return (out, target_edge_weights)


# speedup vs baseline: 11.1603x; 1.1270x over previous
import functools

import jax
import jax.numpy as jnp
from jax import lax
from jax.experimental import pallas as pl
from jax.experimental.pallas import tpu as pltpu
from jax.experimental.pallas import tpu_sc as plsc

N = 10000
D = 128
E_MSG = 256000
E_TGT = 64000
C = 80
NTILES = 16
NPAD = 10240
ROWS_PER_TILE = NPAD // NTILES

_mesh = plsc.VectorSubcoreMesh(core_axis_name="c", subcore_axis_name="s")


def _sc_body(msg_src, msg_dst, tgt_src, tgt_dst, tw, x, zeros, agg_out,
             src_v, dst_v, rows_v, w_v, agg_sh, sem_i, sem_g, sem_s):
    c = lax.axis_index("c")
    s = lax.axis_index("s")
    w = s * 2 + c

    pltpu.sync_copy(zeros, agg_sh.at[pl.ds(s * ROWS_PER_TILE, ROWS_PER_TILE)])
    plsc.subcore_barrier()

    def run_seg(srcs, dsts, n, base, weighted):
        def idx_descs(j):
            b = (base + j) * C
            slot = j % 8
            ds = [pltpu.make_async_copy(srcs.at[pl.ds(b, C)],
                                        src_v.at[slot], sem_i.at[slot]),
                  pltpu.make_async_copy(dsts.at[pl.ds(b, C)],
                                        dst_v.at[slot], sem_i.at[slot])]
            if weighted:
                ds.append(pltpu.make_async_copy(tw.at[pl.ds(b, C)],
                                                w_v.at[slot], sem_i.at[slot]))
            return ds

        def gather_desc(j):
            return pltpu.make_async_copy(x.at[src_v.at[j % 8]],
                                         rows_v.at[j % 4], sem_g.at[j % 4])

        def scatter_desc(j):
            return pltpu.make_async_copy(rows_v.at[j % 4],
                                         agg_sh.at[dst_v.at[j % 8]],
                                         sem_s.at[j % 4])

        def scale(j):
            rbuf = j % 4
            wbuf = j % 8

            def scale_body(e, carry):
                wv = w_v[wbuf, pl.ds((e // 16) * 16, 16)]
                oh = lax.iota(jnp.int32, 16) == (e % 16)
                ws = jnp.broadcast_to(jnp.sum(jnp.where(oh, wv, 0.0)), (16,))
                for k in range(D // 16):
                    rows_v[rbuf, e, pl.ds(k * 16, 16)] = (
                        rows_v[rbuf, e, pl.ds(k * 16, 16)] * ws)
                return carry
            lax.fori_loop(0, C, scale_body, 0)

        for k in range(4):
            for d in idx_descs(k):
                d.start()
        for k in range(2):
            for d in idx_descs(k):
                d.wait()
            gather_desc(k).start()

        def body(j, carry):
            gather_desc(j).wait()

            @pl.when(j >= 2)
            def _():
                scatter_desc(j - 2).wait()

            @pl.when(j + 2 < n)
            def _():
                for d in idx_descs(j + 2):
                    d.wait()
                gather_desc(j + 2).start()

            @pl.when(j + 4 < n)
            def _():
                for d in idx_descs(j + 4):
                    d.start()

            if weighted:
                scale(j)
            scatter_desc(j).start(add=True)
            return carry

        lax.fori_loop(0, n, body, 0)
        scatter_desc(n - 2).wait()
        scatter_desc(n - 1).wait()

    run_seg(msg_src, msg_dst, 100, w * 100, weighted=False)
    run_seg(tgt_src, tgt_dst, 25, w * 25, weighted=True)

    plsc.subcore_barrier()
    pltpu.sync_copy(agg_sh.at[pl.ds(s * ROWS_PER_TILE, ROWS_PER_TILE)],
                    agg_out.at[c, pl.ds(s * ROWS_PER_TILE, ROWS_PER_TILE)])


_sc_scatter = functools.partial(
    pl.kernel,
    out_type=jax.ShapeDtypeStruct((2, NPAD, D), jnp.float32),
    mesh=_mesh,
    compiler_params=pltpu.CompilerParams(needs_layout_passes=False),
    scratch_types=[
        pltpu.VMEM((8, C), jnp.int32),
        pltpu.VMEM((8, C), jnp.int32),
        pltpu.VMEM((4, C, D), jnp.float32),
        pltpu.VMEM((8, C), jnp.float32),
        pltpu.VMEM_SHARED((NPAD, D), jnp.float32),
        pltpu.SemaphoreType.DMA((8,)),
        pltpu.SemaphoreType.DMA((4,)),
        pltpu.SemaphoreType.DMA((4,)),
    ],
)(_sc_body)


ROWS_PER_BLK = 1000


def _dense_body(agg_ref, x_ref, wrel_ref, wroot_ref, wmu_ref,
                brel_ref, bmu_ref, out_ref):
    agg = agg_ref[0] + agg_ref[1]
    conv = jnp.dot(agg, wrel_ref[...], preferred_element_type=jnp.float32)
    conv = conv + jnp.dot(x_ref[...], wroot_ref[...],
                          preferred_element_type=jnp.float32)
    conv = conv + brel_ref[...]
    h = jnp.maximum(conv, 0.0)
    mu = jnp.dot(h, wmu_ref[...], preferred_element_type=jnp.float32)
    mu = mu + bmu_ref[...]
    out_ref[...] = jnp.maximum(mu, 0.0)


_dense = pl.pallas_call(
    _dense_body,
    grid=(N // ROWS_PER_BLK,),
    in_specs=[
        pl.BlockSpec((2, ROWS_PER_BLK, D), lambda i: (0, i, 0)),
        pl.BlockSpec((ROWS_PER_BLK, D), lambda i: (i, 0)),
        pl.BlockSpec((D, D), lambda i: (0, 0)),
        pl.BlockSpec((D, D), lambda i: (0, 0)),
        pl.BlockSpec((D, D), lambda i: (0, 0)),
        pl.BlockSpec((1, D), lambda i: (0, 0)),
        pl.BlockSpec((1, D), lambda i: (0, 0)),
    ],
    out_specs=pl.BlockSpec((ROWS_PER_BLK, D), lambda i: (i, 0)),
    out_shape=jax.ShapeDtypeStruct((N, D), jnp.float32),
)


def kernel(x, message_edge_index, target_edge_index, target_edge_weights,
           W_rel, b_rel, W_root, W_mu, b_mu, W_std, b_std):
    zeros = jnp.zeros((ROWS_PER_TILE, D), jnp.float32)
    agg2 = _sc_scatter(message_edge_index[0], message_edge_index[1],
                       target_edge_index[1], target_edge_index[0],
                       target_edge_weights, x, zeros)
    out = _dense(agg2, x, W_rel, W_root, W_mu,
                 b_rel.reshape(1, D), b_mu.reshape(1, D))
    return (out, target_edge_weights)


# speedup vs baseline: 11.4784x; 1.0285x over previous
import functools

import jax
import jax.numpy as jnp
from jax import lax
from jax.experimental import pallas as pl
from jax.experimental.pallas import tpu as pltpu
from jax.experimental.pallas import tpu_sc as plsc

N = 10000
D = 128
E_MSG = 256000
E_TGT = 64000
C = 80
NTILES = 16
NPAD = 10240
ROWS_PER_TILE = NPAD // NTILES

_mesh = plsc.VectorSubcoreMesh(core_axis_name="c", subcore_axis_name="s")


def _sc_body(msg_src, msg_dst, tgt_src, tgt_dst, tw, x, agg_out,
             src_v, dst_v, rows_v, w_v, agg_sh, sem_i, sem_g, sem_s):
    c = lax.axis_index("c")
    s = lax.axis_index("s")
    w = s * 2 + c

    def zrow(i, carry):
        for k in range(D // 16):
            rows_v[0, i, pl.ds(k * 16, 16)] = jnp.zeros((16,), jnp.float32)
        return carry
    lax.fori_loop(0, C, zrow, 0)
    for k in range(ROWS_PER_TILE // C):
        pltpu.sync_copy(rows_v.at[0],
                        agg_sh.at[pl.ds(s * ROWS_PER_TILE + k * C, C)])
    plsc.subcore_barrier()

    def run_seg(srcs, dsts, n, base, weighted):
        def idx_descs(j):
            b = (base + j) * C
            slot = j % 8
            ds = [pltpu.make_async_copy(srcs.at[pl.ds(b, C)],
                                        src_v.at[slot], sem_i.at[slot]),
                  pltpu.make_async_copy(dsts.at[pl.ds(b, C)],
                                        dst_v.at[slot], sem_i.at[slot])]
            if weighted:
                ds.append(pltpu.make_async_copy(tw.at[pl.ds(b, C)],
                                                w_v.at[slot], sem_i.at[slot]))
            return ds

        def gather_desc(j):
            return pltpu.make_async_copy(x.at[src_v.at[j % 8]],
                                         rows_v.at[j % 4], sem_g.at[j % 4])

        def scatter_desc(j):
            return pltpu.make_async_copy(rows_v.at[j % 4],
                                         agg_sh.at[dst_v.at[j % 8]],
                                         sem_s.at[j % 4])

        def scale(j):
            rbuf = j % 4
            wbuf = j % 8

            def scale_body(e, carry):
                wv = w_v[wbuf, pl.ds((e // 16) * 16, 16)]
                oh = lax.iota(jnp.int32, 16) == (e % 16)
                ws = jnp.broadcast_to(jnp.sum(jnp.where(oh, wv, 0.0)), (16,))
                for k in range(D // 16):
                    rows_v[rbuf, e, pl.ds(k * 16, 16)] = (
                        rows_v[rbuf, e, pl.ds(k * 16, 16)] * ws)
                return carry
            lax.fori_loop(0, C, scale_body, 0)

        for k in range(4):
            for d in idx_descs(k):
                d.start()
        for k in range(2):
            for d in idx_descs(k):
                d.wait()
            gather_desc(k).start()

        def body(j, carry):
            gather_desc(j).wait()

            @pl.when(j >= 2)
            def _():
                scatter_desc(j - 2).wait()

            @pl.when(j + 2 < n)
            def _():
                for d in idx_descs(j + 2):
                    d.wait()
                gather_desc(j + 2).start()

            @pl.when(j + 4 < n)
            def _():
                for d in idx_descs(j + 4):
                    d.start()

            if weighted:
                scale(j)
            scatter_desc(j).start(add=True)
            return carry

        lax.fori_loop(0, n, body, 0)
        scatter_desc(n - 2).wait()
        scatter_desc(n - 1).wait()

    run_seg(msg_src, msg_dst, 100, w * 100, weighted=False)
    run_seg(tgt_src, tgt_dst, 25, w * 25, weighted=True)

    plsc.subcore_barrier()
    pltpu.sync_copy(agg_sh.at[pl.ds(s * ROWS_PER_TILE, ROWS_PER_TILE)],
                    agg_out.at[c, pl.ds(s * ROWS_PER_TILE, ROWS_PER_TILE)])


_sc_scatter = functools.partial(
    pl.kernel,
    out_type=jax.ShapeDtypeStruct((2, NPAD, D), jnp.float32),
    mesh=_mesh,
    compiler_params=pltpu.CompilerParams(needs_layout_passes=False),
    scratch_types=[
        pltpu.VMEM((8, C), jnp.int32),
        pltpu.VMEM((8, C), jnp.int32),
        pltpu.VMEM((4, C, D), jnp.float32),
        pltpu.VMEM((8, C), jnp.float32),
        pltpu.VMEM_SHARED((NPAD, D), jnp.float32),
        pltpu.SemaphoreType.DMA((8,)),
        pltpu.SemaphoreType.DMA((4,)),
        pltpu.SemaphoreType.DMA((4,)),
    ],
)(_sc_body)


ROWS_PER_BLK = 1000


def _root_body(x_ref, wroot_ref, brel_ref, root_ref):
    root_ref[...] = jnp.dot(x_ref[...], wroot_ref[...],
                            preferred_element_type=jnp.float32) + brel_ref[...]


_root = pl.pallas_call(
    _root_body,
    grid=(N // ROWS_PER_BLK,),
    in_specs=[
        pl.BlockSpec((ROWS_PER_BLK, D), lambda i: (i, 0)),
        pl.BlockSpec((D, D), lambda i: (0, 0)),
        pl.BlockSpec((1, D), lambda i: (0, 0)),
    ],
    out_specs=pl.BlockSpec((ROWS_PER_BLK, D), lambda i: (i, 0)),
    out_shape=jax.ShapeDtypeStruct((N, D), jnp.float32),
)


def _dense_body(agg_ref, root_ref, wrel_ref, wmu_ref, bmu_ref, out_ref):
    agg = agg_ref[0] + agg_ref[1]
    conv = jnp.dot(agg, wrel_ref[...], preferred_element_type=jnp.float32)
    h = jnp.maximum(conv + root_ref[...], 0.0)
    mu = jnp.dot(h, wmu_ref[...], preferred_element_type=jnp.float32)
    out_ref[...] = jnp.maximum(mu + bmu_ref[...], 0.0)


_dense = pl.pallas_call(
    _dense_body,
    grid=(N // ROWS_PER_BLK,),
    in_specs=[
        pl.BlockSpec((2, ROWS_PER_BLK, D), lambda i: (0, i, 0)),
        pl.BlockSpec((ROWS_PER_BLK, D), lambda i: (i, 0)),
        pl.BlockSpec((D, D), lambda i: (0, 0)),
        pl.BlockSpec((D, D), lambda i: (0, 0)),
        pl.BlockSpec((1, D), lambda i: (0, 0)),
    ],
    out_specs=pl.BlockSpec((ROWS_PER_BLK, D), lambda i: (i, 0)),
    out_shape=jax.ShapeDtypeStruct((N, D), jnp.float32),
)


def kernel(x, message_edge_index, target_edge_index, target_edge_weights,
           W_rel, b_rel, W_root, W_mu, b_mu, W_std, b_std):
    root = _root(x, W_root, b_rel.reshape(1, D))
    agg2 = _sc_scatter(message_edge_index[0], message_edge_index[1],
                       target_edge_index[1], target_edge_index[0],
                       target_edge_weights, x)
    out = _dense(agg2, root, W_rel, W_mu, b_mu.reshape(1, D))
    return (out, target_edge_weights)
